# Initial kernel scaffold; baseline (speedup 1.0000x reference)
import jax
import jax.numpy as jnp
from jax.experimental import pallas as pl


def kernel(x, edge_index, labels, actions, w_n2l, bias_n2l, conv_w, conv_b, le1_w, le1_b, le2_w, le2_b, l1_w, l1_b, lout_w, lout_b):
    raise NotImplementedError("write your pallas kernel here")


# speedup vs baseline: 7.5751x; 7.5751x over previous
import functools

import jax
import jax.numpy as jnp
from jax import lax
from jax.experimental import pallas as pl
from jax.experimental.pallas import tpu as pltpu
from jax.experimental.pallas import tpu_sc as plsc

_N = 10000
_E = 320000
_D = 128
_H = 64
_C = 16
_NI = 10
_A = 1000
_MLP = 64

_NC = 2
_NS = 16
_NW = _NC * _NS
_CHUNK = 128
_NCHUNK = _E // _CHUNK
_STRIPE = 624
_STRIPE_LAST = _N - (_NS - 1) * _STRIPE
_APAD = 1024


def _embed_body(x_ref, w_ref, b_ref, msg_ref, emb_ref):
    m = jnp.dot(x_ref[...], w_ref[...], preferred_element_type=jnp.float32)
    m = m + b_ref[...]
    msg_ref[...] = m
    emb_ref[...] = jnp.maximum(m, 0.0)


def _embed(x, w, b):
    return pl.pallas_call(
        _embed_body,
        out_shape=(
            jax.ShapeDtypeStruct((_N, _H), jnp.float32),
            jax.ShapeDtypeStruct((_N, _H), jnp.float32),
        ),
    )(x, w, b)


def _edge_body(emb_hbm, src_hbm, dst_hbm, act_hbm, acc_out, deg_out, mask_out,
               src_idx, dst_idx, rows_v, deg_v, mask_v, acc_sh, sem):
    c = lax.axis_index("c")
    s = lax.axis_index("s")
    wid = s * _NC + c

    z16 = jnp.zeros((16,), jnp.float32)
    ones16 = jnp.ones((16,), jnp.float32)

    @pl.loop(0, _CHUNK)
    def _zr(i):
        @pl.loop(0, _H // 16)
        def _zc(j):
            rows_v[i, pl.ds(j * 16, 16)] = z16

    base = s * _STRIPE

    @pl.when(s != _NS - 1)
    def _za():
        for k in range(_STRIPE // _CHUNK):
            pltpu.sync_copy(rows_v,
                            acc_sh.at[pl.ds(base + k * _CHUNK, _CHUNK)])
        rem = _STRIPE % _CHUNK
        pltpu.sync_copy(
            rows_v.at[pl.ds(0, rem)],
            acc_sh.at[pl.ds(base + (_STRIPE // _CHUNK) * _CHUNK, rem)])

    @pl.when(s == _NS - 1)
    def _zb():
        lb = (_NS - 1) * _STRIPE
        for k in range(_STRIPE_LAST // _CHUNK):
            pltpu.sync_copy(rows_v, acc_sh.at[pl.ds(lb + k * _CHUNK, _CHUNK)])

    @pl.loop(0, _N // 16)
    def _zd(i):
        deg_v[pl.ds(i * 16, 16)] = z16

    plsc.subcore_barrier()

    n_j = (_NCHUNK - wid + _NW - 1) // _NW

    @pl.loop(0, n_j)
    def _edge(j):
        off = (wid + j * _NW) * _CHUNK
        pltpu.sync_copy(src_hbm.at[pl.ds(off, _CHUNK)], src_idx)
        pltpu.sync_copy(dst_hbm.at[pl.ds(off, _CHUNK)], dst_idx.at[0])
        pltpu.async_copy(emb_hbm.at[src_idx], rows_v, sem).wait()
        pltpu.sync_copy(rows_v, acc_sh.at[dst_idx.at[0]], add=True)
        for q in range(_CHUNK // 16):
            d16 = dst_idx[0, pl.ds(q * 16, 16)]
            plsc.addupdate_scatter(deg_v, [d16], ones16)

    @pl.when(wid == 0)
    def _mask():
        @pl.loop(0, _N // 16)
        def _zm(i):
            mask_v[pl.ds(i * 16, 16)] = z16

        for a_off in range(0, _APAD, _CHUNK):
            pltpu.sync_copy(act_hbm.at[pl.ds(a_off, _CHUNK)], dst_idx.at[0])
            for q in range(_CHUNK // 16):
                a16 = dst_idx[0, pl.ds(q * 16, 16)]
                plsc.addupdate_scatter(mask_v, [a16], ones16)
        pltpu.sync_copy(mask_v, mask_out.at[0])

    pltpu.sync_copy(deg_v, deg_out.at[wid, 0])

    plsc.subcore_barrier()

    @pl.when(s != _NS - 1)
    def _ca():
        for k in range(_STRIPE // _CHUNK):
            o = base + k * _CHUNK
            pltpu.sync_copy(acc_sh.at[pl.ds(o, _CHUNK)], rows_v)
            pltpu.sync_copy(rows_v, acc_out.at[c, pl.ds(o, _CHUNK)])
        rem = _STRIPE % _CHUNK
        o = base + (_STRIPE // _CHUNK) * _CHUNK
        pltpu.sync_copy(acc_sh.at[pl.ds(o, rem)], rows_v.at[pl.ds(0, rem)])
        pltpu.sync_copy(rows_v.at[pl.ds(0, rem)],
                        acc_out.at[c, pl.ds(o, rem)])

    @pl.when(s == _NS - 1)
    def _cb():
        lb = (_NS - 1) * _STRIPE
        for k in range(_STRIPE_LAST // _CHUNK):
            o = lb + k * _CHUNK
            pltpu.sync_copy(acc_sh.at[pl.ds(o, _CHUNK)], rows_v)
            pltpu.sync_copy(rows_v, acc_out.at[c, pl.ds(o, _CHUNK)])


def _edge_kernel_fn():
    mesh = plsc.VectorSubcoreMesh(core_axis_name="c", subcore_axis_name="s",
                                  num_cores=_NC, num_subcores=_NS)

    return pl.kernel(
        _edge_body,
        out_type=(
            jax.ShapeDtypeStruct((_NC, _N, _H), jnp.float32),
            jax.ShapeDtypeStruct((_NW, 1, _N), jnp.float32),
            jax.ShapeDtypeStruct((1, _N), jnp.float32),
        ),
        mesh=mesh,
        compiler_params=pltpu.CompilerParams(needs_layout_passes=False,
                                             use_tc_tiling_on_sc=False),
        scratch_types=(
            pltpu.VMEM((_CHUNK,), jnp.int32),
            pltpu.VMEM((1, _CHUNK), jnp.int32),
            pltpu.VMEM((_CHUNK, _H), jnp.float32),
            pltpu.VMEM((_N,), jnp.float32),
            pltpu.VMEM((_N,), jnp.float32),
            pltpu.VMEM_SHARED((_N, _H), jnp.float32),
            pltpu.SemaphoreType.DMA,
        ),
    )


def _final_body(labels_ref, acc_ref, degp_ref, mask_ref, msg_ref,
                convw_ref, convb_ref, le1w_ref, le1b_ref, le2w_ref, le2b_ref,
                l1w_ref, l1b_ref, loutw_ref, loutb_ref, out_ref):
    f32 = jnp.float32
    accs = acc_ref[0] + acc_ref[1]
    deg = lax.dot_general(degp_ref[...], jnp.ones((_NW, 1), f32),
                          (((0,), (0,)), ((), ())),
                          preferred_element_type=f32)
    deg = jnp.maximum(deg, 1.0)
    n2npool = accs / deg
    node_linear = jnp.dot(n2npool, convw_ref[...], preferred_element_type=f32)
    ne2 = jnp.maximum(node_linear + convb_ref[...] + msg_ref[...], 0.0)

    graph = jnp.sum(ne2, axis=0, keepdims=True) * (1.0 / _N)

    col = lax.broadcasted_iota(jnp.int32, (1, _C * _NI), 1)
    oh = jnp.zeros((1, _C * _NI), f32)
    for i in range(_NI):
        oh = oh + (col == (i * _C + labels_ref[i])).astype(f32)
    h1 = jnp.dot(oh, le1w_ref[...], preferred_element_type=f32) + le1b_ref[...]
    h1 = jnp.maximum(h1, 0.0)
    lemb = jnp.dot(h1, le2w_ref[...], preferred_element_type=f32) + le2b_ref[...]
    lemb = jnp.maximum(lemb, 0.0)

    base = (jnp.dot(graph, l1w_ref[0:_H, :], preferred_element_type=f32)
            + jnp.dot(lemb, l1w_ref[_H:2 * _H, :], preferred_element_type=f32)
            + l1b_ref[...])
    z = jnp.dot(ne2, l1w_ref[2 * _H:3 * _H, :], preferred_element_type=f32)
    hid = jnp.maximum(z + base, 0.0)
    scores = (jnp.dot(hid, loutw_ref[...], preferred_element_type=f32)
              + loutb_ref[...])

    maskc = lax.dot_general(mask_ref[...], jnp.ones((1, 1), f32),
                            (((0,), (0,)), ((), ())),
                            preferred_element_type=f32)
    pred = jnp.max(jnp.where(maskc > 0.0, scores, -jnp.inf))
    out_ref[...] = pred.reshape(1, 1)


def _final(labels, acc, degp, maskr, msg, convw, convb, le1w, le1b, le2w,
           le2b, l1w, l1b, loutw, loutb):
    specs = [pl.BlockSpec(memory_space=pltpu.SMEM)] + [
        pl.BlockSpec(memory_space=pltpu.VMEM) for _ in range(14)
    ]
    return pl.pallas_call(
        _final_body,
        in_specs=specs,
        out_shape=jax.ShapeDtypeStruct((1, 1), jnp.float32),
    )(labels, acc, degp, maskr, msg, convw, convb, le1w, le1b, le2w, le2b,
      l1w, l1b, loutw, loutb)


def kernel(x, edge_index, labels, actions, w_n2l, bias_n2l, conv_w, conv_b,
           le1_w, le1_b, le2_w, le2_b, l1_w, l1_b, lout_w, lout_b):
    msg, emb = _embed(x, w_n2l, bias_n2l.reshape(1, _H))
    act_pad = jnp.concatenate([actions, actions[:_APAD - _A]])
    acc, degp, maskr = _edge_kernel_fn()(emb, edge_index[0], edge_index[1],
                                         act_pad)
    degp = degp.reshape(_NW, _N)
    pred = _final(labels, acc, degp, maskr, msg,
                  conv_w, conv_b.reshape(1, _H),
                  le1_w, le1_b.reshape(1, _MLP),
                  le2_w, le2_b.reshape(1, _H),
                  l1_w, l1_b.reshape(1, _MLP),
                  lout_w, lout_b.reshape(1, 1))
    return pred.reshape(())


# speedup vs baseline: 10.3456x; 1.3657x over previous
import functools

import jax
import jax.numpy as jnp
from jax import lax
from jax.experimental import pallas as pl
from jax.experimental.pallas import tpu as pltpu
from jax.experimental.pallas import tpu_sc as plsc

_N = 10000
_E = 320000
_D = 128
_H = 64
_C = 16
_NI = 10
_A = 1000
_MLP = 64

_NC = 2
_NS = 16
_NW = _NC * _NS
_CHUNK = 128
_NCHUNK = _E // _CHUNK
_STRIPE = 624
_STRIPE_LAST = _N - (_NS - 1) * _STRIPE
_APAD = 1024


def _embed_body(x_ref, w_ref, b_ref, msg_ref, emb_ref):
    m = jnp.dot(x_ref[...], w_ref[...], preferred_element_type=jnp.float32)
    m = m + b_ref[...]
    msg_ref[...] = m
    emb_ref[...] = jnp.maximum(m, 0.0)


def _embed(x, w, b):
    return pl.pallas_call(
        _embed_body,
        out_shape=(
            jax.ShapeDtypeStruct((_N, _H), jnp.float32),
            jax.ShapeDtypeStruct((_N, _H), jnp.float32),
        ),
    )(x, w, b)


def _edge_body(emb_hbm, src_hbm, dst_hbm, act_hbm, acc_out, deg_out, mask_out,
               src_idx, dst_idx, rows_v, deg_v, mask_v, acc_sh, sem, sem2):
    c = lax.axis_index("c")
    s = lax.axis_index("s")
    wid = s * _NC + c

    z16 = jnp.zeros((16,), jnp.float32)
    ones16 = jnp.ones((16,), jnp.float32)

    @pl.loop(0, _CHUNK)
    def _zr(i):
        @pl.loop(0, _H // 16)
        def _zc(j):
            rows_v[0, i, pl.ds(j * 16, 16)] = z16

    base = s * _STRIPE

    @pl.when(s != _NS - 1)
    def _za():
        for k in range(_STRIPE // _CHUNK):
            pltpu.sync_copy(rows_v.at[0],
                            acc_sh.at[pl.ds(base + k * _CHUNK, _CHUNK)])
        rem = _STRIPE % _CHUNK
        pltpu.sync_copy(
            rows_v.at[0, pl.ds(0, rem)],
            acc_sh.at[pl.ds(base + (_STRIPE // _CHUNK) * _CHUNK, rem)])

    @pl.when(s == _NS - 1)
    def _zb():
        lb = (_NS - 1) * _STRIPE
        for k in range(_STRIPE_LAST // _CHUNK):
            pltpu.sync_copy(rows_v.at[0],
                            acc_sh.at[pl.ds(lb + k * _CHUNK, _CHUNK)])

    @pl.loop(0, _N // 16)
    def _zd(i):
        deg_v[pl.ds(i * 16, 16)] = z16

    plsc.subcore_barrier()

    _NJ = 2496 // _NW

    def _start(b, j):
        off = (wid + j * _NW) * _CHUNK
        pltpu.sync_copy(src_hbm.at[pl.ds(off, _CHUNK)], src_idx.at[b])
        pltpu.sync_copy(dst_hbm.at[pl.ds(off, _CHUNK)], dst_idx.at[b])
        pltpu.async_copy(emb_hbm.at[src_idx.at[b]], rows_v.at[b], sems[b])

    def _drain(b):
        pltpu.make_async_copy(emb_hbm.at[src_idx.at[b]], rows_v.at[b],
                              sems[b]).wait()
        pltpu.sync_copy(rows_v.at[b], acc_sh.at[dst_idx.at[b]], add=True)
        for q in range(_CHUNK // 16):
            d16 = dst_idx[b, pl.ds(q * 16, 16)]
            plsc.addupdate_scatter(deg_v, [d16], ones16)

    sems = (sem, sem2)
    for b in range(2):
        _start(b, b)

    @pl.loop(0, (_NJ - 2) // 2)
    def _edge(jp):
        for b in range(2):
            _drain(b)
            _start(b, 2 * jp + b + 2)

    for b in range(2):
        _drain(b)

    @pl.when(wid < _NCHUNK - _NJ * _NW)
    def _leftover():
        off = (_NJ * _NW + wid) * _CHUNK
        pltpu.sync_copy(src_hbm.at[pl.ds(off, _CHUNK)], src_idx.at[0])
        pltpu.sync_copy(dst_hbm.at[pl.ds(off, _CHUNK)], dst_idx.at[0])
        pltpu.async_copy(emb_hbm.at[src_idx.at[0]], rows_v.at[0], sem).wait()
        pltpu.sync_copy(rows_v.at[0], acc_sh.at[dst_idx.at[0]], add=True)
        for q in range(_CHUNK // 16):
            d16 = dst_idx[0, pl.ds(q * 16, 16)]
            plsc.addupdate_scatter(deg_v, [d16], ones16)

    @pl.when(wid == 0)
    def _mask():
        @pl.loop(0, _N // 16)
        def _zm(i):
            mask_v[pl.ds(i * 16, 16)] = z16

        for a_off in range(0, _APAD, _CHUNK):
            pltpu.sync_copy(act_hbm.at[pl.ds(a_off, _CHUNK)], dst_idx.at[0])
            for q in range(_CHUNK // 16):
                a16 = dst_idx[0, pl.ds(q * 16, 16)]
                plsc.addupdate_scatter(mask_v, [a16], ones16)
        pltpu.sync_copy(mask_v, mask_out.at[0])

    pltpu.sync_copy(deg_v, deg_out.at[wid, 0])

    plsc.subcore_barrier()

    @pl.when(s != _NS - 1)
    def _ca():
        for k in range(_STRIPE // _CHUNK):
            o = base + k * _CHUNK
            pltpu.sync_copy(acc_sh.at[pl.ds(o, _CHUNK)], rows_v.at[0])
            pltpu.sync_copy(rows_v.at[0], acc_out.at[c, pl.ds(o, _CHUNK)])
        rem = _STRIPE % _CHUNK
        o = base + (_STRIPE // _CHUNK) * _CHUNK
        pltpu.sync_copy(acc_sh.at[pl.ds(o, rem)], rows_v.at[0, pl.ds(0, rem)])
        pltpu.sync_copy(rows_v.at[0, pl.ds(0, rem)],
                        acc_out.at[c, pl.ds(o, rem)])

    @pl.when(s == _NS - 1)
    def _cb():
        lb = (_NS - 1) * _STRIPE
        for k in range(_STRIPE_LAST // _CHUNK):
            o = lb + k * _CHUNK
            pltpu.sync_copy(acc_sh.at[pl.ds(o, _CHUNK)], rows_v.at[0])
            pltpu.sync_copy(rows_v.at[0], acc_out.at[c, pl.ds(o, _CHUNK)])


def _edge_kernel_fn():
    mesh = plsc.VectorSubcoreMesh(core_axis_name="c", subcore_axis_name="s",
                                  num_cores=_NC, num_subcores=_NS)

    return pl.kernel(
        _edge_body,
        out_type=(
            jax.ShapeDtypeStruct((_NC, _N, _H), jnp.float32),
            jax.ShapeDtypeStruct((_NW, 1, _N), jnp.float32),
            jax.ShapeDtypeStruct((1, _N), jnp.float32),
        ),
        mesh=mesh,
        compiler_params=pltpu.CompilerParams(needs_layout_passes=False,
                                             use_tc_tiling_on_sc=False),
        scratch_types=(
            pltpu.VMEM((2, _CHUNK), jnp.int32),
            pltpu.VMEM((2, _CHUNK), jnp.int32),
            pltpu.VMEM((2, _CHUNK, _H), jnp.float32),
            pltpu.VMEM((_N,), jnp.float32),
            pltpu.VMEM((_N,), jnp.float32),
            pltpu.VMEM_SHARED((_N, _H), jnp.float32),
            pltpu.SemaphoreType.DMA,
            pltpu.SemaphoreType.DMA,
        ),
    )


def _final_body(labels_ref, acc_ref, degp_ref, mask_ref, msg_ref,
                convw_ref, convb_ref, le1w_ref, le1b_ref, le2w_ref, le2b_ref,
                l1w_ref, l1b_ref, loutw_ref, loutb_ref, out_ref):
    f32 = jnp.float32
    accs = acc_ref[0] + acc_ref[1]
    deg = lax.dot_general(degp_ref[...], jnp.ones((_NW, 1), f32),
                          (((0,), (0,)), ((), ())),
                          preferred_element_type=f32)
    deg = jnp.maximum(deg, 1.0)
    n2npool = accs / deg
    node_linear = jnp.dot(n2npool, convw_ref[...], preferred_element_type=f32)
    ne2 = jnp.maximum(node_linear + convb_ref[...] + msg_ref[...], 0.0)

    graph = jnp.sum(ne2, axis=0, keepdims=True) * (1.0 / _N)

    col = lax.broadcasted_iota(jnp.int32, (1, _C * _NI), 1)
    oh = jnp.zeros((1, _C * _NI), f32)
    for i in range(_NI):
        oh = oh + (col == (i * _C + labels_ref[i])).astype(f32)
    h1 = jnp.dot(oh, le1w_ref[...], preferred_element_type=f32) + le1b_ref[...]
    h1 = jnp.maximum(h1, 0.0)
    lemb = jnp.dot(h1, le2w_ref[...], preferred_element_type=f32) + le2b_ref[...]
    lemb = jnp.maximum(lemb, 0.0)

    base = (jnp.dot(graph, l1w_ref[0:_H, :], preferred_element_type=f32)
            + jnp.dot(lemb, l1w_ref[_H:2 * _H, :], preferred_element_type=f32)
            + l1b_ref[...])
    z = jnp.dot(ne2, l1w_ref[2 * _H:3 * _H, :], preferred_element_type=f32)
    hid = jnp.maximum(z + base, 0.0)
    scores = (jnp.dot(hid, loutw_ref[...], preferred_element_type=f32)
              + loutb_ref[...])

    maskc = lax.dot_general(mask_ref[...], jnp.ones((1, 1), f32),
                            (((0,), (0,)), ((), ())),
                            preferred_element_type=f32)
    pred = jnp.max(jnp.where(maskc > 0.0, scores, -jnp.inf))
    out_ref[...] = pred.reshape(1, 1)


def _final(labels, acc, degp, maskr, msg, convw, convb, le1w, le1b, le2w,
           le2b, l1w, l1b, loutw, loutb):
    specs = [pl.BlockSpec(memory_space=pltpu.SMEM)] + [
        pl.BlockSpec(memory_space=pltpu.VMEM) for _ in range(14)
    ]
    return pl.pallas_call(
        _final_body,
        in_specs=specs,
        out_shape=jax.ShapeDtypeStruct((1, 1), jnp.float32),
    )(labels, acc, degp, maskr, msg, convw, convb, le1w, le1b, le2w, le2b,
      l1w, l1b, loutw, loutb)


def kernel(x, edge_index, labels, actions, w_n2l, bias_n2l, conv_w, conv_b,
           le1_w, le1_b, le2_w, le2_b, l1_w, l1_b, lout_w, lout_b):
    msg, emb = _embed(x, w_n2l, bias_n2l.reshape(1, _H))
    act_pad = jnp.concatenate([actions, actions[:_APAD - _A]])
    acc, degp, maskr = _edge_kernel_fn()(emb, edge_index[0], edge_index[1],
                                         act_pad)
    degp = degp.reshape(_NW, _N)
    pred = _final(labels, acc, degp, maskr, msg,
                  conv_w, conv_b.reshape(1, _H),
                  le1_w, le1_b.reshape(1, _MLP),
                  le2_w, le2_b.reshape(1, _H),
                  l1_w, l1_b.reshape(1, _MLP),
                  lout_w, lout_b.reshape(1, 1))
    return pred.reshape(())


# speedup vs baseline: 10.4714x; 1.0122x over previous
import functools

import jax
import jax.numpy as jnp
from jax import lax
from jax.experimental import pallas as pl
from jax.experimental.pallas import tpu as pltpu
from jax.experimental.pallas import tpu_sc as plsc

_N = 10000
_E = 320000
_D = 128
_H = 64
_C = 16
_NI = 10
_A = 1000
_MLP = 64

_NC = 2
_NS = 16
_NW = _NC * _NS
_CHUNK = 128
_NCHUNK = _E // _CHUNK
_STRIPE = 624
_STRIPE_LAST = _N - (_NS - 1) * _STRIPE
_APAD = 1024


def _embed_body(x_ref, w_ref, b_ref, msg_ref, emb_ref):
    m = jnp.dot(x_ref[...], w_ref[...], preferred_element_type=jnp.float32)
    m = m + b_ref[...]
    msg_ref[...] = m
    emb_ref[...] = jnp.maximum(m, 0.0)


def _embed(x, w, b):
    return pl.pallas_call(
        _embed_body,
        out_shape=(
            jax.ShapeDtypeStruct((_N, _H), jnp.float32),
            jax.ShapeDtypeStruct((_N, _H), jnp.float32),
        ),
    )(x, w, b)


def _edge_body(emb_hbm, src_hbm, dst_hbm, act_hbm, acc_out, deg_out, mask_out,
               src_idx, dst_idx, rows_v, deg_v, mask_v, acc_sh,
               sem, sem2, sem3):
    c = lax.axis_index("c")
    s = lax.axis_index("s")
    wid = s * _NC + c

    z16 = jnp.zeros((16,), jnp.float32)
    ones16 = jnp.ones((16,), jnp.float32)

    @pl.loop(0, _CHUNK)
    def _zr(i):
        @pl.loop(0, _H // 16)
        def _zc(j):
            rows_v[0, i, pl.ds(j * 16, 16)] = z16

    base = s * _STRIPE

    @pl.when(s != _NS - 1)
    def _za():
        for k in range(_STRIPE // _CHUNK):
            pltpu.sync_copy(rows_v.at[0],
                            acc_sh.at[pl.ds(base + k * _CHUNK, _CHUNK)])
        rem = _STRIPE % _CHUNK
        pltpu.sync_copy(
            rows_v.at[0, pl.ds(0, rem)],
            acc_sh.at[pl.ds(base + (_STRIPE // _CHUNK) * _CHUNK, rem)])

    @pl.when(s == _NS - 1)
    def _zb():
        lb = (_NS - 1) * _STRIPE
        for k in range(_STRIPE_LAST // _CHUNK):
            pltpu.sync_copy(rows_v.at[0],
                            acc_sh.at[pl.ds(lb + k * _CHUNK, _CHUNK)])

    @pl.loop(0, _N // 16)
    def _zd(i):
        deg_v[pl.ds(i * 16, 16)] = z16

    plsc.subcore_barrier()

    _K = 3
    _NJ = 2496 // _NW
    _NG = _NJ // _K
    sem_s = (sem2, sem3)

    def _fire_group(p, g):
        for b in range(_K):
            slot = p * _K + b
            off = (wid + (g * _K + b) * _NW) * _CHUNK
            pltpu.sync_copy(src_hbm.at[pl.ds(off, _CHUNK)], src_idx.at[slot])
            pltpu.sync_copy(dst_hbm.at[pl.ds(off, _CHUNK)], dst_idx.at[slot])
            pltpu.async_copy(emb_hbm.at[src_idx.at[slot]], rows_v.at[slot],
                             sem)

    def _drain_gathers(p):
        for b in range(_K):
            slot = p * _K + b
            pltpu.make_async_copy(emb_hbm.at[src_idx.at[slot]],
                                  rows_v.at[slot], sem).wait()

    def _fire_scatters(p):
        for b in range(_K):
            slot = p * _K + b
            pltpu.async_copy(rows_v.at[slot], acc_sh.at[dst_idx.at[slot]],
                             sem_s[p], add=True)

    def _deg_group(p):
        for b in range(_K):
            slot = p * _K + b
            for q in range(_CHUNK // 16):
                d16 = dst_idx[slot, pl.ds(q * 16, 16)]
                plsc.addupdate_scatter(deg_v, [d16], ones16)

    def _drain_scatters(p):
        for b in range(_K):
            slot = p * _K + b
            pltpu.make_async_copy(rows_v.at[slot],
                                  acc_sh.at[dst_idx.at[slot]],
                                  sem_s[p]).wait()

    _fire_group(0, 0)

    @pl.loop(0, _NG // 2)
    def _edge(gp):
        _drain_gathers(0)
        _fire_scatters(0)
        _deg_group(0)

        @pl.when(gp > 0)
        def _ds1():
            _drain_scatters(1)

        _fire_group(1, 2 * gp + 1)
        _drain_gathers(1)
        _fire_scatters(1)
        _deg_group(1)
        _drain_scatters(0)

        @pl.when(gp < _NG // 2 - 1)
        def _fg0():
            _fire_group(0, 2 * gp + 2)

    _drain_scatters(1)

    @pl.when(wid < _NCHUNK - _NJ * _NW)
    def _leftover():
        off = (_NJ * _NW + wid) * _CHUNK
        pltpu.sync_copy(src_hbm.at[pl.ds(off, _CHUNK)], src_idx.at[0])
        pltpu.sync_copy(dst_hbm.at[pl.ds(off, _CHUNK)], dst_idx.at[0])
        pltpu.async_copy(emb_hbm.at[src_idx.at[0]], rows_v.at[0], sem).wait()
        pltpu.sync_copy(rows_v.at[0], acc_sh.at[dst_idx.at[0]], add=True)
        for q in range(_CHUNK // 16):
            d16 = dst_idx[0, pl.ds(q * 16, 16)]
            plsc.addupdate_scatter(deg_v, [d16], ones16)

    @pl.when(wid == 0)
    def _mask():
        @pl.loop(0, _N // 16)
        def _zm(i):
            mask_v[pl.ds(i * 16, 16)] = z16

        for a_off in range(0, _APAD, _CHUNK):
            pltpu.sync_copy(act_hbm.at[pl.ds(a_off, _CHUNK)], dst_idx.at[0])
            for q in range(_CHUNK // 16):
                a16 = dst_idx[0, pl.ds(q * 16, 16)]
                plsc.addupdate_scatter(mask_v, [a16], ones16)
        pltpu.sync_copy(mask_v, mask_out.at[0])

    pltpu.sync_copy(deg_v, deg_out.at[wid, 0])

    plsc.subcore_barrier()

    @pl.when(s != _NS - 1)
    def _ca():
        for k in range(_STRIPE // _CHUNK):
            o = base + k * _CHUNK
            pltpu.sync_copy(acc_sh.at[pl.ds(o, _CHUNK)], rows_v.at[0])
            pltpu.sync_copy(rows_v.at[0], acc_out.at[c, pl.ds(o, _CHUNK)])
        rem = _STRIPE % _CHUNK
        o = base + (_STRIPE // _CHUNK) * _CHUNK
        pltpu.sync_copy(acc_sh.at[pl.ds(o, rem)], rows_v.at[0, pl.ds(0, rem)])
        pltpu.sync_copy(rows_v.at[0, pl.ds(0, rem)],
                        acc_out.at[c, pl.ds(o, rem)])

    @pl.when(s == _NS - 1)
    def _cb():
        lb = (_NS - 1) * _STRIPE
        for k in range(_STRIPE_LAST // _CHUNK):
            o = lb + k * _CHUNK
            pltpu.sync_copy(acc_sh.at[pl.ds(o, _CHUNK)], rows_v.at[0])
            pltpu.sync_copy(rows_v.at[0], acc_out.at[c, pl.ds(o, _CHUNK)])


def _edge_kernel_fn():
    mesh = plsc.VectorSubcoreMesh(core_axis_name="c", subcore_axis_name="s",
                                  num_cores=_NC, num_subcores=_NS)

    return pl.kernel(
        _edge_body,
        out_type=(
            jax.ShapeDtypeStruct((_NC, _N, _H), jnp.float32),
            jax.ShapeDtypeStruct((_NW, 1, _N), jnp.float32),
            jax.ShapeDtypeStruct((1, _N), jnp.float32),
        ),
        mesh=mesh,
        compiler_params=pltpu.CompilerParams(needs_layout_passes=False,
                                             use_tc_tiling_on_sc=False),
        scratch_types=(
            pltpu.VMEM((6, _CHUNK), jnp.int32),
            pltpu.VMEM((6, _CHUNK), jnp.int32),
            pltpu.VMEM((6, _CHUNK, _H), jnp.float32),
            pltpu.VMEM((_N,), jnp.float32),
            pltpu.VMEM((_N,), jnp.float32),
            pltpu.VMEM_SHARED((_N, _H), jnp.float32),
            pltpu.SemaphoreType.DMA,
            pltpu.SemaphoreType.DMA,
            pltpu.SemaphoreType.DMA,
        ),
    )


def _final_body(labels_ref, acc_ref, degp_ref, mask_ref, msg_ref,
                convw_ref, convb_ref, le1w_ref, le1b_ref, le2w_ref, le2b_ref,
                l1w_ref, l1b_ref, loutw_ref, loutb_ref, out_ref):
    f32 = jnp.float32
    accs = acc_ref[0] + acc_ref[1]
    deg = lax.dot_general(degp_ref[...], jnp.ones((_NW, 1), f32),
                          (((0,), (0,)), ((), ())),
                          preferred_element_type=f32)
    deg = jnp.maximum(deg, 1.0)
    n2npool = accs / deg
    node_linear = jnp.dot(n2npool, convw_ref[...], preferred_element_type=f32)
    ne2 = jnp.maximum(node_linear + convb_ref[...] + msg_ref[...], 0.0)

    graph = jnp.sum(ne2, axis=0, keepdims=True) * (1.0 / _N)

    col = lax.broadcasted_iota(jnp.int32, (1, _C * _NI), 1)
    oh = jnp.zeros((1, _C * _NI), f32)
    for i in range(_NI):
        oh = oh + (col == (i * _C + labels_ref[i])).astype(f32)
    h1 = jnp.dot(oh, le1w_ref[...], preferred_element_type=f32) + le1b_ref[...]
    h1 = jnp.maximum(h1, 0.0)
    lemb = jnp.dot(h1, le2w_ref[...], preferred_element_type=f32) + le2b_ref[...]
    lemb = jnp.maximum(lemb, 0.0)

    base = (jnp.dot(graph, l1w_ref[0:_H, :], preferred_element_type=f32)
            + jnp.dot(lemb, l1w_ref[_H:2 * _H, :], preferred_element_type=f32)
            + l1b_ref[...])
    z = jnp.dot(ne2, l1w_ref[2 * _H:3 * _H, :], preferred_element_type=f32)
    hid = jnp.maximum(z + base, 0.0)
    scores = (jnp.dot(hid, loutw_ref[...], preferred_element_type=f32)
              + loutb_ref[...])

    maskc = lax.dot_general(mask_ref[...], jnp.ones((1, 1), f32),
                            (((0,), (0,)), ((), ())),
                            preferred_element_type=f32)
    pred = jnp.max(jnp.where(maskc > 0.0, scores, -jnp.inf))
    out_ref[...] = pred.reshape(1, 1)


def _final(labels, acc, degp, maskr, msg, convw, convb, le1w, le1b, le2w,
           le2b, l1w, l1b, loutw, loutb):
    specs = [pl.BlockSpec(memory_space=pltpu.SMEM)] + [
        pl.BlockSpec(memory_space=pltpu.VMEM) for _ in range(14)
    ]
    return pl.pallas_call(
        _final_body,
        in_specs=specs,
        out_shape=jax.ShapeDtypeStruct((1, 1), jnp.float32),
    )(labels, acc, degp, maskr, msg, convw, convb, le1w, le1b, le2w, le2b,
      l1w, l1b, loutw, loutb)


def kernel(x, edge_index, labels, actions, w_n2l, bias_n2l, conv_w, conv_b,
           le1_w, le1_b, le2_w, le2_b, l1_w, l1_b, lout_w, lout_b):
    msg, emb = _embed(x, w_n2l, bias_n2l.reshape(1, _H))
    act_pad = jnp.concatenate([actions, actions[:_APAD - _A]])
    acc, degp, maskr = _edge_kernel_fn()(emb, edge_index[0], edge_index[1],
                                         act_pad)
    degp = degp.reshape(_NW, _N)
    pred = _final(labels, acc, degp, maskr, msg,
                  conv_w, conv_b.reshape(1, _H),
                  le1_w, le1_b.reshape(1, _MLP),
                  le2_w, le2_b.reshape(1, _H),
                  l1_w, l1_b.reshape(1, _MLP),
                  lout_w, lout_b.reshape(1, 1))
    return pred.reshape(())


# speedup vs baseline: 14.7312x; 1.4068x over previous
import functools

import jax
import jax.numpy as jnp
from jax import lax
from jax.experimental import pallas as pl
from jax.experimental.pallas import tpu as pltpu
from jax.experimental.pallas import tpu_sc as plsc

_N = 10000
_E = 320000
_D = 128
_H = 64
_C = 16
_NI = 10
_A = 1000
_MLP = 64

_NC = 2
_NS = 16
_NW = _NC * _NS
_CHUNK = 128
_NCHUNK = _E // _CHUNK
_STRIPE = 624
_STRIPE_LAST = _N - (_NS - 1) * _STRIPE
_APAD = 1024


def _embed_body(x_ref, w_ref, b_ref, msg_ref, emb_ref):
    m = jnp.dot(x_ref[...], w_ref[...], preferred_element_type=jnp.float32)
    m = m + b_ref[...]
    msg_ref[...] = m
    emb_ref[...] = jnp.maximum(m, 0.0)


def _embed(x, w, b):
    return pl.pallas_call(
        _embed_body,
        out_shape=(
            jax.ShapeDtypeStruct((_N, _H), jnp.float32),
            jax.ShapeDtypeStruct((_N, _H), jnp.float32),
        ),
    )(x, w, b)


def _edge_body(emb_hbm, src_hbm, dst_hbm, act_hbm, acc_out, deg_out, mask_out,
               src_all, dst_all, rows_v, deg_v, mask_v, acc_sh,
               sem, sem2, sem3):
    c = lax.axis_index("c")
    s = lax.axis_index("s")
    wid = s * _NC + c

    z16 = jnp.zeros((16,), jnp.float32)
    ones16 = jnp.ones((16,), jnp.float32)

    _NJ = 2496 // _NW

    idx_src_cp = pltpu.async_copy(src_hbm.at[pl.ds(wid * _NJ, _NJ)],
                                  src_all, sem2)
    idx_dst_cp = pltpu.async_copy(dst_hbm.at[pl.ds(wid * _NJ, _NJ)],
                                  dst_all, sem3)

    @pl.loop(0, _CHUNK)
    def _zr(i):
        @pl.loop(0, _H // 16)
        def _zc(j):
            rows_v[0, i, pl.ds(j * 16, 16)] = z16

    base = s * _STRIPE

    @pl.when(s != _NS - 1)
    def _za():
        for k in range(_STRIPE // _CHUNK):
            pltpu.sync_copy(rows_v.at[0],
                            acc_sh.at[pl.ds(base + k * _CHUNK, _CHUNK)])
        rem = _STRIPE % _CHUNK
        pltpu.sync_copy(
            rows_v.at[0, pl.ds(0, rem)],
            acc_sh.at[pl.ds(base + (_STRIPE // _CHUNK) * _CHUNK, rem)])

    @pl.when(s == _NS - 1)
    def _zb():
        lb = (_NS - 1) * _STRIPE
        for k in range(_STRIPE_LAST // _CHUNK):
            pltpu.sync_copy(rows_v.at[0],
                            acc_sh.at[pl.ds(lb + k * _CHUNK, _CHUNK)])

    @pl.loop(0, _N // 16)
    def _zd(i):
        deg_v[pl.ds(i * 16, 16)] = z16

    idx_src_cp.wait()
    idx_dst_cp.wait()

    plsc.subcore_barrier()

    _K = 3
    _NG = _NJ // _K
    sem_s = (sem2, sem3)

    def _fire_group(p, g):
        for b in range(_K):
            slot = p * _K + b
            pltpu.async_copy(emb_hbm.at[src_all.at[g * _K + b]],
                             rows_v.at[slot], sem)

    def _drain_gathers(p):
        for b in range(_K):
            slot = p * _K + b
            pltpu.make_async_copy(emb_hbm.at[src_all.at[0]],
                                  rows_v.at[slot], sem).wait()

    def _fire_scatters(p, g):
        for b in range(_K):
            slot = p * _K + b
            pltpu.async_copy(rows_v.at[slot],
                             acc_sh.at[dst_all.at[g * _K + b]],
                             sem_s[p], add=True)

    def _deg_group(g):
        for b in range(_K):
            j = g * _K + b
            for q in range(_CHUNK // 16):
                d16 = dst_all[j, pl.ds(q * 16, 16)]
                plsc.addupdate_scatter(deg_v, [d16], ones16)

    def _drain_scatters(p):
        for b in range(_K):
            slot = p * _K + b
            pltpu.make_async_copy(rows_v.at[slot],
                                  acc_sh.at[dst_all.at[0]],
                                  sem_s[p]).wait()

    _fire_group(0, 0)

    @pl.loop(0, _NG // 2)
    def _edge(gp):
        _drain_gathers(0)
        _fire_scatters(0, 2 * gp)
        _deg_group(2 * gp)

        @pl.when(gp > 0)
        def _ds1():
            _drain_scatters(1)

        _fire_group(1, 2 * gp + 1)
        _drain_gathers(1)
        _fire_scatters(1, 2 * gp + 1)
        _deg_group(2 * gp + 1)
        _drain_scatters(0)

        @pl.when(gp < _NG // 2 - 1)
        def _fg0():
            _fire_group(0, 2 * gp + 2)

    _drain_scatters(1)

    @pl.when(wid < _NCHUNK - _NJ * _NW)
    def _leftover():
        ch = _NJ * _NW + wid
        pltpu.sync_copy(src_hbm.at[ch], src_all.at[0])
        pltpu.sync_copy(dst_hbm.at[ch], dst_all.at[0])
        pltpu.async_copy(emb_hbm.at[src_all.at[0]], rows_v.at[0], sem).wait()
        pltpu.sync_copy(rows_v.at[0], acc_sh.at[dst_all.at[0]], add=True)
        for q in range(_CHUNK // 16):
            d16 = dst_all[0, pl.ds(q * 16, 16)]
            plsc.addupdate_scatter(deg_v, [d16], ones16)

    @pl.when(wid < _APAD // _CHUNK)
    def _mask():
        @pl.loop(0, _N // 16)
        def _zm(i):
            mask_v[pl.ds(i * 16, 16)] = z16

        pltpu.sync_copy(act_hbm.at[wid], dst_all.at[0])
        for q in range(_CHUNK // 16):
            a16 = dst_all[0, pl.ds(q * 16, 16)]
            plsc.addupdate_scatter(mask_v, [a16], ones16)
        pltpu.sync_copy(mask_v, mask_out.at[wid])

    pltpu.sync_copy(deg_v, deg_out.at[wid, 0])

    plsc.subcore_barrier()

    @pl.when(s != _NS - 1)
    def _ca():
        for k in range(_STRIPE // _CHUNK):
            o = base + k * _CHUNK
            pltpu.sync_copy(acc_sh.at[pl.ds(o, _CHUNK)], rows_v.at[0])
            pltpu.sync_copy(rows_v.at[0], acc_out.at[c, pl.ds(o, _CHUNK)])
        rem = _STRIPE % _CHUNK
        o = base + (_STRIPE // _CHUNK) * _CHUNK
        pltpu.sync_copy(acc_sh.at[pl.ds(o, rem)], rows_v.at[0, pl.ds(0, rem)])
        pltpu.sync_copy(rows_v.at[0, pl.ds(0, rem)],
                        acc_out.at[c, pl.ds(o, rem)])

    @pl.when(s == _NS - 1)
    def _cb():
        lb = (_NS - 1) * _STRIPE
        for k in range(_STRIPE_LAST // _CHUNK):
            o = lb + k * _CHUNK
            pltpu.sync_copy(acc_sh.at[pl.ds(o, _CHUNK)], rows_v.at[0])
            pltpu.sync_copy(rows_v.at[0], acc_out.at[c, pl.ds(o, _CHUNK)])


def _edge_kernel_fn():
    mesh = plsc.VectorSubcoreMesh(core_axis_name="c", subcore_axis_name="s",
                                  num_cores=_NC, num_subcores=_NS)

    return pl.kernel(
        _edge_body,
        out_type=(
            jax.ShapeDtypeStruct((_NC, _N, _H), jnp.float32),
            jax.ShapeDtypeStruct((_NW, 1, _N), jnp.float32),
            jax.ShapeDtypeStruct((_APAD // _CHUNK, _N), jnp.float32),
        ),
        mesh=mesh,
        compiler_params=pltpu.CompilerParams(needs_layout_passes=False,
                                             use_tc_tiling_on_sc=False),
        scratch_types=(
            pltpu.VMEM((2496 // _NW, _CHUNK), jnp.int32),
            pltpu.VMEM((2496 // _NW, _CHUNK), jnp.int32),
            pltpu.VMEM((6, _CHUNK, _H), jnp.float32),
            pltpu.VMEM((_N,), jnp.float32),
            pltpu.VMEM((_N,), jnp.float32),
            pltpu.VMEM_SHARED((_N, _H), jnp.float32),
            pltpu.SemaphoreType.DMA,
            pltpu.SemaphoreType.DMA,
            pltpu.SemaphoreType.DMA,
        ),
    )


def _final_body(labels_ref, acc_ref, degp_ref, mask_ref, msg_ref,
                convw_ref, convb_ref, le1w_ref, le1b_ref, le2w_ref, le2b_ref,
                l1w_ref, l1b_ref, loutw_ref, loutb_ref, out_ref):
    f32 = jnp.float32
    accs = acc_ref[0] + acc_ref[1]
    deg = lax.dot_general(degp_ref[...], jnp.ones((_NW, 1), f32),
                          (((0,), (0,)), ((), ())),
                          preferred_element_type=f32)
    deg = jnp.maximum(deg, 1.0)
    n2npool = accs / deg
    node_linear = jnp.dot(n2npool, convw_ref[...], preferred_element_type=f32)
    ne2 = jnp.maximum(node_linear + convb_ref[...] + msg_ref[...], 0.0)

    graph = jnp.sum(ne2, axis=0, keepdims=True) * (1.0 / _N)

    col = lax.broadcasted_iota(jnp.int32, (1, _C * _NI), 1)
    oh = jnp.zeros((1, _C * _NI), f32)
    for i in range(_NI):
        oh = oh + (col == (i * _C + labels_ref[i])).astype(f32)
    h1 = jnp.dot(oh, le1w_ref[...], preferred_element_type=f32) + le1b_ref[...]
    h1 = jnp.maximum(h1, 0.0)
    lemb = jnp.dot(h1, le2w_ref[...], preferred_element_type=f32) + le2b_ref[...]
    lemb = jnp.maximum(lemb, 0.0)

    base = (jnp.dot(graph, l1w_ref[0:_H, :], preferred_element_type=f32)
            + jnp.dot(lemb, l1w_ref[_H:2 * _H, :], preferred_element_type=f32)
            + l1b_ref[...])
    z = jnp.dot(ne2, l1w_ref[2 * _H:3 * _H, :], preferred_element_type=f32)
    hid = jnp.maximum(z + base, 0.0)
    scores = (jnp.dot(hid, loutw_ref[...], preferred_element_type=f32)
              + loutb_ref[...])

    maskc = lax.dot_general(mask_ref[...], jnp.ones((_APAD // _CHUNK, 1), f32),
                            (((0,), (0,)), ((), ())),
                            preferred_element_type=f32)
    pred = jnp.max(jnp.where(maskc > 0.0, scores, -jnp.inf))
    out_ref[...] = pred.reshape(1, 1)


def _final(labels, acc, degp, maskr, msg, convw, convb, le1w, le1b, le2w,
           le2b, l1w, l1b, loutw, loutb):
    specs = [pl.BlockSpec(memory_space=pltpu.SMEM)] + [
        pl.BlockSpec(memory_space=pltpu.VMEM) for _ in range(14)
    ]
    return pl.pallas_call(
        _final_body,
        in_specs=specs,
        out_shape=jax.ShapeDtypeStruct((1, 1), jnp.float32),
    )(labels, acc, degp, maskr, msg, convw, convb, le1w, le1b, le2w, le2b,
      l1w, l1b, loutw, loutb)


def kernel(x, edge_index, labels, actions, w_n2l, bias_n2l, conv_w, conv_b,
           le1_w, le1_b, le2_w, le2_b, l1_w, l1_b, lout_w, lout_b):
    msg, emb = _embed(x, w_n2l, bias_n2l.reshape(1, _H))
    act_pad = jnp.concatenate([actions, actions[:_APAD - _A]])
    acc, degp, maskr = _edge_kernel_fn()(
        emb,
        edge_index[0].reshape(_NCHUNK, _CHUNK),
        edge_index[1].reshape(_NCHUNK, _CHUNK),
        act_pad.reshape(_APAD // _CHUNK, _CHUNK))
    degp = degp.reshape(_NW, _N)
    pred = _final(labels, acc, degp, maskr, msg,
                  conv_w, conv_b.reshape(1, _H),
                  le1_w, le1_b.reshape(1, _MLP),
                  le2_w, le2_b.reshape(1, _H),
                  l1_w, l1_b.reshape(1, _MLP),
                  lout_w, lout_b.reshape(1, 1))
    return pred.reshape(())


# speedup vs baseline: 15.8215x; 1.0740x over previous
import functools

import jax
import jax.numpy as jnp
from jax import lax
from jax.experimental import pallas as pl
from jax.experimental.pallas import tpu as pltpu
from jax.experimental.pallas import tpu_sc as plsc

_N = 10000
_E = 320000
_D = 128
_H = 64
_C = 16
_NI = 10
_A = 1000
_MLP = 64

_NC = 2
_NS = 16
_NW = _NC * _NS
_CHUNK = 128
_NCHUNK = _E // _CHUNK
_STRIPE = 624
_STRIPE_LAST = _N - (_NS - 1) * _STRIPE
_NMASKW = 8


def _embed_body(x_ref, w_ref, b_ref, msg_ref, emb_ref):
    m = jnp.dot(x_ref[...], w_ref[...], preferred_element_type=jnp.float32)
    m = m + b_ref[...]
    msg_ref[...] = m
    emb_ref[...] = jnp.maximum(m, 0.0)


def _embed(x, w, b):
    return pl.pallas_call(
        _embed_body,
        out_shape=(
            jax.ShapeDtypeStruct((_N, _H), jnp.float32),
            jax.ShapeDtypeStruct((_N, _H), jnp.float32),
        ),
    )(x, w, b)


def _edge_body(emb_hbm, edge_hbm, act_hbm, acc_out, deg_out, mask_out,
               src_all, dst_all, rows_v, deg_v, mask_v, acc_sh,
               sem, sem2, sem3):
    c = lax.axis_index("c")
    s = lax.axis_index("s")
    wid = s * _NC + c

    z16 = jnp.zeros((16,), jnp.float32)
    ones16 = jnp.ones((16,), jnp.float32)

    _NJ = 2496 // _NW
    ebase = wid * _NJ * _CHUNK

    idx_src_cp = pltpu.async_copy(
        edge_hbm.at[0, pl.ds(ebase, _NJ * _CHUNK)], src_all, sem2)

    @pl.loop(0, _NJ)
    def _ld(j):
        pltpu.async_copy(edge_hbm.at[1, pl.ds(ebase + j * _CHUNK, _CHUNK)],
                         dst_all.at[j], sem3)

    @pl.loop(0, _CHUNK)
    def _zr(i):
        @pl.loop(0, _H // 16)
        def _zc(j):
            rows_v[0, i, pl.ds(j * 16, 16)] = z16

    base = s * _STRIPE

    @pl.when(s != _NS - 1)
    def _za():
        for k in range(_STRIPE // _CHUNK):
            pltpu.sync_copy(rows_v.at[0],
                            acc_sh.at[pl.ds(base + k * _CHUNK, _CHUNK)])
        rem = _STRIPE % _CHUNK
        pltpu.sync_copy(
            rows_v.at[0, pl.ds(0, rem)],
            acc_sh.at[pl.ds(base + (_STRIPE // _CHUNK) * _CHUNK, rem)])

    @pl.when(s == _NS - 1)
    def _zb():
        lb = (_NS - 1) * _STRIPE
        for k in range(_STRIPE_LAST // _CHUNK):
            pltpu.sync_copy(rows_v.at[0],
                            acc_sh.at[pl.ds(lb + k * _CHUNK, _CHUNK)])

    @pl.loop(0, _N // 16)
    def _zd(i):
        deg_v[pl.ds(i * 16, 16)] = z16

    idx_src_cp.wait()

    @pl.loop(0, _NJ)
    def _ldw(j):
        pltpu.make_async_copy(edge_hbm.at[1, pl.ds(ebase, _CHUNK)],
                              dst_all.at[j], sem3).wait()

    plsc.subcore_barrier()

    _K = 3
    _NG = _NJ // _K
    sem_s = (sem2, sem3)

    def _fire_group(p, g):
        for b in range(_K):
            slot = p * _K + b
            pltpu.async_copy(
                emb_hbm.at[src_all.at[pl.ds((g * _K + b) * _CHUNK, _CHUNK)]],
                rows_v.at[slot], sem)

    def _drain_gathers(p):
        for b in range(_K):
            slot = p * _K + b
            pltpu.make_async_copy(emb_hbm.at[src_all.at[pl.ds(0, _CHUNK)]],
                                  rows_v.at[slot], sem).wait()

    def _fire_scatters(p, g):
        for b in range(_K):
            slot = p * _K + b
            pltpu.async_copy(rows_v.at[slot],
                             acc_sh.at[dst_all.at[g * _K + b]],
                             sem_s[p], add=True)

    def _deg_group(g):
        for b in range(_K):
            j = g * _K + b
            for q in range(_CHUNK // 16):
                d16 = dst_all[j, pl.ds(q * 16, 16)]
                plsc.addupdate_scatter(deg_v, [d16], ones16)

    def _drain_scatters(p):
        for b in range(_K):
            slot = p * _K + b
            pltpu.make_async_copy(rows_v.at[slot],
                                  acc_sh.at[dst_all.at[0]],
                                  sem_s[p]).wait()

    _fire_group(0, 0)

    @pl.loop(0, _NG // 2)
    def _edge(gp):
        _drain_gathers(0)
        _fire_scatters(0, 2 * gp)
        _deg_group(2 * gp)

        @pl.when(gp > 0)
        def _ds1():
            _drain_scatters(1)

        _fire_group(1, 2 * gp + 1)
        _drain_gathers(1)
        _fire_scatters(1, 2 * gp + 1)
        _deg_group(2 * gp + 1)
        _drain_scatters(0)

        @pl.when(gp < _NG // 2 - 1)
        def _fg0():
            _fire_group(0, 2 * gp + 2)

    _drain_scatters(1)

    @pl.when(wid < _NCHUNK - _NJ * _NW)
    def _leftover():
        off = (_NJ * _NW + wid) * _CHUNK
        pltpu.sync_copy(edge_hbm.at[0, pl.ds(off, _CHUNK)],
                        src_all.at[pl.ds(0, _CHUNK)])
        pltpu.sync_copy(edge_hbm.at[1, pl.ds(off, _CHUNK)], dst_all.at[0])
        pltpu.async_copy(emb_hbm.at[src_all.at[pl.ds(0, _CHUNK)]],
                         rows_v.at[0], sem).wait()
        pltpu.sync_copy(rows_v.at[0], acc_sh.at[dst_all.at[0]], add=True)
        for q in range(_CHUNK // 16):
            d16 = dst_all[0, pl.ds(q * 16, 16)]
            plsc.addupdate_scatter(deg_v, [d16], ones16)

    @pl.when(wid < _NMASKW)
    def _mask():
        @pl.loop(0, _N // 16)
        def _zm(i):
            mask_v[pl.ds(i * 16, 16)] = z16

        a_off = jnp.where(wid == _NMASKW - 1, _A - _CHUNK, wid * _CHUNK)
        pltpu.sync_copy(act_hbm.at[pl.ds(a_off, _CHUNK)], dst_all.at[0])
        for q in range(_CHUNK // 16):
            a16 = dst_all[0, pl.ds(q * 16, 16)]
            plsc.addupdate_scatter(mask_v, [a16], ones16)
        pltpu.sync_copy(mask_v, mask_out.at[wid])

    pltpu.sync_copy(deg_v, deg_out.at[wid, 0])

    plsc.subcore_barrier()

    @pl.when(s != _NS - 1)
    def _ca():
        for k in range(_STRIPE // _CHUNK):
            o = base + k * _CHUNK
            pltpu.sync_copy(acc_sh.at[pl.ds(o, _CHUNK)], rows_v.at[0])
            pltpu.sync_copy(rows_v.at[0], acc_out.at[c, pl.ds(o, _CHUNK)])
        rem = _STRIPE % _CHUNK
        o = base + (_STRIPE // _CHUNK) * _CHUNK
        pltpu.sync_copy(acc_sh.at[pl.ds(o, rem)], rows_v.at[0, pl.ds(0, rem)])
        pltpu.sync_copy(rows_v.at[0, pl.ds(0, rem)],
                        acc_out.at[c, pl.ds(o, rem)])

    @pl.when(s == _NS - 1)
    def _cb():
        lb = (_NS - 1) * _STRIPE
        for k in range(_STRIPE_LAST // _CHUNK):
            o = lb + k * _CHUNK
            pltpu.sync_copy(acc_sh.at[pl.ds(o, _CHUNK)], rows_v.at[0])
            pltpu.sync_copy(rows_v.at[0], acc_out.at[c, pl.ds(o, _CHUNK)])


def _edge_kernel_fn():
    mesh = plsc.VectorSubcoreMesh(core_axis_name="c", subcore_axis_name="s",
                                  num_cores=_NC, num_subcores=_NS)

    return pl.kernel(
        _edge_body,
        out_type=(
            jax.ShapeDtypeStruct((_NC, _N, _H), jnp.float32),
            jax.ShapeDtypeStruct((_NW, 1, _N), jnp.float32),
            jax.ShapeDtypeStruct((_NMASKW, _N), jnp.float32),
        ),
        mesh=mesh,
        compiler_params=pltpu.CompilerParams(needs_layout_passes=False,
                                             use_tc_tiling_on_sc=False),
        scratch_types=(
            pltpu.VMEM((2496 // _NW * _CHUNK,), jnp.int32),
            pltpu.VMEM((2496 // _NW, _CHUNK), jnp.int32),
            pltpu.VMEM((6, _CHUNK, _H), jnp.float32),
            pltpu.VMEM((_N,), jnp.float32),
            pltpu.VMEM((_N,), jnp.float32),
            pltpu.VMEM_SHARED((_N, _H), jnp.float32),
            pltpu.SemaphoreType.DMA,
            pltpu.SemaphoreType.DMA,
            pltpu.SemaphoreType.DMA,
        ),
    )


def _final_body(labels_ref, acc_ref, degp_ref, mask_ref, msg_ref,
                convw_ref, convb_ref, le1w_ref, le1b_ref, le2w_ref, le2b_ref,
                l1w_ref, l1b_ref, loutw_ref, loutb_ref, out_ref):
    f32 = jnp.float32
    accs = acc_ref[0] + acc_ref[1]
    deg = lax.dot_general(degp_ref[...], jnp.ones((_NW, 1), f32),
                          (((0,), (0,)), ((), ())),
                          preferred_element_type=f32)
    deg = jnp.maximum(deg, 1.0)
    n2npool = accs / deg
    node_linear = jnp.dot(n2npool, convw_ref[...], preferred_element_type=f32)
    ne2 = jnp.maximum(node_linear + convb_ref[...] + msg_ref[...], 0.0)

    graph = jnp.sum(ne2, axis=0, keepdims=True) * (1.0 / _N)

    col = lax.broadcasted_iota(jnp.int32, (1, _C * _NI), 1)
    oh = jnp.zeros((1, _C * _NI), f32)
    for i in range(_NI):
        oh = oh + (col == (i * _C + labels_ref[i])).astype(f32)
    h1 = jnp.dot(oh, le1w_ref[...], preferred_element_type=f32) + le1b_ref[...]
    h1 = jnp.maximum(h1, 0.0)
    lemb = jnp.dot(h1, le2w_ref[...], preferred_element_type=f32) + le2b_ref[...]
    lemb = jnp.maximum(lemb, 0.0)

    base = (jnp.dot(graph, l1w_ref[0:_H, :], preferred_element_type=f32)
            + jnp.dot(lemb, l1w_ref[_H:2 * _H, :], preferred_element_type=f32)
            + l1b_ref[...])
    z = jnp.dot(ne2, l1w_ref[2 * _H:3 * _H, :], preferred_element_type=f32)
    hid = jnp.maximum(z + base, 0.0)
    scores = (jnp.dot(hid, loutw_ref[...], preferred_element_type=f32)
              + loutb_ref[...])

    maskc = lax.dot_general(mask_ref[...], jnp.ones((_NMASKW, 1), f32),
                            (((0,), (0,)), ((), ())),
                            preferred_element_type=f32)
    pred = jnp.max(jnp.where(maskc > 0.0, scores, -jnp.inf))
    out_ref[...] = pred.reshape(1, 1)


def _final(labels, acc, degp, maskr, msg, convw, convb, le1w, le1b, le2w,
           le2b, l1w, l1b, loutw, loutb):
    specs = [pl.BlockSpec(memory_space=pltpu.SMEM)] + [
        pl.BlockSpec(memory_space=pltpu.VMEM) for _ in range(14)
    ]
    return pl.pallas_call(
        _final_body,
        in_specs=specs,
        out_shape=jax.ShapeDtypeStruct((1, 1), jnp.float32),
    )(labels, acc, degp, maskr, msg, convw, convb, le1w, le1b, le2w, le2b,
      l1w, l1b, loutw, loutb)


def kernel(x, edge_index, labels, actions, w_n2l, bias_n2l, conv_w, conv_b,
           le1_w, le1_b, le2_w, le2_b, l1_w, l1_b, lout_w, lout_b):
    msg, emb = _embed(x, w_n2l, bias_n2l.reshape(1, _H))
    acc, degp, maskr = _edge_kernel_fn()(emb, edge_index, actions)
    degp = degp.reshape(_NW, _N)
    pred = _final(labels, acc, degp, maskr, msg,
                  conv_w, conv_b.reshape(1, _H),
                  le1_w, le1_b.reshape(1, _MLP),
                  le2_w, le2_b.reshape(1, _H),
                  l1_w, l1_b.reshape(1, _MLP),
                  lout_w, lout_b.reshape(1, 1))
    return pred.reshape(())
